# Initial kernel scaffold; baseline (speedup 1.0000x reference)
import jax
import jax.numpy as jnp
from jax.experimental import pallas as pl


def kernel(x, edge_index, edge_attr, W, b):
    raise NotImplementedError("write your pallas kernel here")


# speedup vs baseline: 4.5576x; 4.5576x over previous
import jax
import jax.numpy as jnp
from jax import lax
from jax.experimental import pallas as pl
from jax.experimental.pallas import tpu as pltpu
from jax.experimental.pallas import tpu_sc as plsc

N = 100000
E = 3200000
C = 10
ED = 4
ACCEL_SCALE = 0.02
MAX_VEL = 0.02

LANE = 16
ROWW = 128
ROWS = E // ROWW
NC, NS, NW = 2, 16, 32
WA = 21
RPT_A = 784
RPT_B = 776
BASE_B = WA * RPT_A
CHR = 16
NCH = 48
TAILR = 8

NPAD = 100352
NPC = NPAD // NS
NPC_LAST = N - (NS - 1) * NPC
NPT = NPAD // NW
NPT_LAST = N - (NW - 1) * NPT

_MESH = dict(core_axis_name="c", subcore_axis_name="s", num_cores=NC,
             num_subcores=NS)


def _i16(v):
  return jnp.full((LANE,), v, jnp.int32)


def _k1_body(x_hbm, ei_hbm, ea_hbm, w_hbm, b_hbm, hpart,
             p_sh, h0_sh, h1_sh, xb, pb, wv, bv,
             srcb, dstb, eab, prb, msg0, msg1, sem_in, sem_g, sem_s):
  cid = lax.axis_index("c")
  sid = lax.axis_index("s")
  wch = cid * NS + sid

  pltpu.sync_copy(w_hbm, wv.at[pl.ds(0, (C + ED) * 7)])
  pltpu.sync_copy(b_hbm, bv.at[pl.ds(0, 7)])
  wchunks = [wv[pl.ds(16 * i, 16)] for i in range(7)]
  bchunk = bv[pl.ds(0, 16)]

  def _wsc(k, c):
    f = 7 * k + c
    return wchunks[f // 16][f % 16]

  iota = lax.iota(jnp.int32, LANE)
  iotc = iota * C
  iote = iota * ED
  f0 = jnp.zeros((LANE,), jnp.float32)
  nbase = sid * NPC

  def _zfill(i, carry):
    pb[pl.ds(i * LANE, LANE)] = f0
    return carry

  lax.fori_loop(0, NPC // LANE, _zfill, None)
  pltpu.sync_copy(pb.at[pl.ds(0, NPC)], h0_sh.at[pl.ds(nbase, NPC)])
  pltpu.sync_copy(pb.at[pl.ds(0, NPC)], h1_sh.at[pl.ds(nbase, NPC)])

  w0 = [_wsc(k, 0) for k in range(C + ED)]
  w1 = [_wsc(k, 1) for k in range(C + ED)]
  b0 = bchunk[0]
  b1 = bchunk[1]

  def _run_p(nr):
    pltpu.sync_copy(x_hbm.at[pl.ds(nbase * C, nr * C)],
                    xb.at[pl.ds(0, nr * C)])

    def _pg(g, carry):
      acc0 = f0 + b0
      acc1 = f0 + b1
      for k in range(C):
        xk = plsc.load_gather(xb, [iotc + (g * (LANE * C) + k)])
        acc0 = acc0 + xk * w0[k]
        acc1 = acc1 + xk * w1[k]
      packed = plsc.pack(acc0, acc1, format=plsc.PackFormat.INTERLEAVED)
      pb[pl.ds(g * LANE, LANE)] = plsc.bitcast(packed, jnp.float32)
      return carry

    lax.fori_loop(0, nr // LANE, _pg, None)
    pltpu.sync_copy(pb.at[pl.ds(0, nr)], p_sh.at[pl.ds(nbase, nr)])

  @pl.when(sid < NS - 1)
  def _():
    _run_p(NPC)

  @pl.when(sid == NS - 1)
  def _():
    _run_p(NPC_LAST)

  plsc.subcore_barrier()

  def _chunk(crows):
    def run(row0):
      d1 = pltpu.async_copy(ei_hbm.at[0, pl.ds(row0, crows), :],
                            srcb.at[pl.ds(0, crows), :], sem_in)
      d2 = pltpu.async_copy(ei_hbm.at[1, pl.ds(row0, crows), :],
                            dstb.at[pl.ds(0, crows), :], sem_in)
      d3 = pltpu.async_copy(ea_hbm.at[pl.ds(row0 * (ROWW * ED),
                                            crows * ROWW * ED)],
                            eab.at[pl.ds(0, crows * ROWW * ED)], sem_in)
      d1.wait()
      d2.wait()
      d3.wait()
      gd = [pltpu.async_copy(p_sh.at[srcb.at[j]],
                             prb.at[pl.ds(j * ROWW, ROWW)], sem_g)
            for j in range(crows)]
      for d in gd:
        d.wait()

      def _grp(g, carry):
        praw = prb[pl.ds(g * LANE, LANE)]
        pbf = plsc.bitcast(praw, jnp.bfloat16)
        p0, p1 = plsc.unpack(pbf, format=plsc.PackFormat.INTERLEAVED)
        ebase = g * (LANE * ED)
        e0 = plsc.load_gather(eab, [iote + ebase])
        e1 = plsc.load_gather(eab, [iote + (ebase + 1)])
        e2 = plsc.load_gather(eab, [iote + (ebase + 2)])
        e3 = plsc.load_gather(eab, [iote + (ebase + 3)])
        z0 = (p0 + e0 * w0[C] + e1 * w0[C + 1]
              + e2 * w0[C + 2] + e3 * w0[C + 3])
        z1 = (p1 + e0 * w1[C] + e1 * w1[C + 1]
              + e2 * w1[C + 2] + e3 * w1[C + 3])
        t0 = jnp.exp(z0 + z0)
        t1 = jnp.exp(z1 + z1)
        m0 = 1.0 - 2.0 / (t0 + 1.0)
        m1 = 1.0 - 2.0 / (t1 + 1.0)
        msg0[pl.ds(g * LANE, LANE)] = m0
        msg1[pl.ds(g * LANE, LANE)] = m1
        return carry

      lax.fori_loop(0, crows * (ROWW // LANE), _grp, None)
      sd = []
      for j in range(crows):
        sd.append(pltpu.async_copy(msg0.at[pl.ds(j * ROWW, ROWW)],
                                   h0_sh.at[dstb.at[j]], sem_s, add=True))
        sd.append(pltpu.async_copy(msg1.at[pl.ds(j * ROWW, ROWW)],
                                   h1_sh.at[dstb.at[j]], sem_s, add=True))
      for d in sd:
        d.wait()

    return run

  run_main = _chunk(CHR)
  run_tail = _chunk(TAILR)

  is_a = wch < WA
  row_base = jnp.where(is_a, wch * RPT_A, BASE_B + (wch - WA) * RPT_B)

  def _mc(i, carry):
    run_main(row_base + i * CHR)
    return carry

  lax.fori_loop(0, NCH, _mc, None)

  @pl.when(is_a)
  def _():
    run_main(row_base + NCH * CHR)

  @pl.when(jnp.logical_not(is_a))
  def _():
    run_tail(row_base + NCH * CHR)

  plsc.subcore_barrier()

  pltpu.sync_copy(h0_sh.at[pl.ds(nbase, NPC)],
                  hpart.at[cid, 0, pl.ds(nbase, NPC)])
  pltpu.sync_copy(h1_sh.at[pl.ds(nbase, NPC)],
                  hpart.at[cid, 1, pl.ds(nbase, NPC)])


def _k2_body(x_hbm, hp_hbm, out_hbm, ob, h0a, h0b, h1a, h1b):
  cid = lax.axis_index("c")
  sid = lax.axis_index("s")
  wch = cid * NS + sid
  base = wch * NPT

  iota = lax.iota(jnp.int32, LANE)
  iotc = iota * C

  def _run(nr):
    pltpu.sync_copy(x_hbm.at[pl.ds(base * C, nr * C)], ob.at[pl.ds(0, nr * C)])
    pltpu.sync_copy(hp_hbm.at[0, 0, pl.ds(base, nr)], h0a.at[pl.ds(0, nr)])
    pltpu.sync_copy(hp_hbm.at[1, 0, pl.ds(base, nr)], h0b.at[pl.ds(0, nr)])
    pltpu.sync_copy(hp_hbm.at[0, 1, pl.ds(base, nr)], h1a.at[pl.ds(0, nr)])
    pltpu.sync_copy(hp_hbm.at[1, 1, pl.ds(base, nr)], h1b.at[pl.ds(0, nr)])

    def _grp(g, carry):
      rb = g * LANE
      cb = g * (LANE * C)
      x0 = plsc.load_gather(ob, [iotc + cb])
      x1 = plsc.load_gather(ob, [iotc + (cb + 1)])
      x2 = plsc.load_gather(ob, [iotc + (cb + 2)])
      x3 = plsc.load_gather(ob, [iotc + (cb + 3)])
      x4 = plsc.load_gather(ob, [iotc + (cb + 4)])
      h0 = h0a[pl.ds(rb, LANE)] + h0b[pl.ds(rb, LANE)]
      h1 = h1a[pl.ds(rb, LANE)] + h1b[pl.ds(rb, LANE)]
      cm = jnp.where(x4 > 0.0, 1.0, 0.0).astype(jnp.float32)
      a0 = h0 * cm * ACCEL_SCALE
      a1 = h1 * cm * ACCEL_SCALE
      v0 = x2 + a0
      v1 = x3 + a1
      s = v0 * v0 + v1 * v1 + 1e-12
      ii = lax.bitcast_convert_type(s, jnp.int32)
      yi = jnp.int32(0x5F3759DF) - (ii >> 1)
      y = lax.bitcast_convert_type(yi, jnp.float32)
      for _ in range(4):
        y = y * (1.5 - 0.5 * s * y * y)
      factor = jnp.minimum(1.0, MAX_VEL * y)
      v0 = v0 * factor * cm
      v1 = v1 * factor * cm
      t0 = x0 + v0 + 1.0
      t1 = x1 + v1 + 1.0
      r0 = lax.rem(t0, jnp.float32(2.0))
      r1 = lax.rem(t1, jnp.float32(2.0))
      r0 = jnp.where(r0 < 0.0, r0 + 2.0, r0) - 1.0
      r1 = jnp.where(r1 < 0.0, r1 + 2.0, r1) - 1.0
      plsc.store_scatter(ob, [iotc + cb], r0)
      plsc.store_scatter(ob, [iotc + (cb + 1)], r1)
      plsc.store_scatter(ob, [iotc + (cb + 2)], v0)
      plsc.store_scatter(ob, [iotc + (cb + 3)], v1)
      return carry

    lax.fori_loop(0, nr // LANE, _grp, None)
    pltpu.sync_copy(ob.at[pl.ds(0, nr * C)],
                    out_hbm.at[pl.ds(base * C, nr * C)])

  @pl.when(wch < NW - 1)
  def _():
    _run(NPT)

  @pl.when(wch == NW - 1)
  def _():
    _run(NPT_LAST)


def _make_kernels():
  mesh = plsc.VectorSubcoreMesh(**_MESH)
  cparams = pltpu.CompilerParams(needs_layout_passes=False,
                                 use_tc_tiling_on_sc=False)
  k1 = pl.kernel(
      _k1_body,
      out_type=jax.ShapeDtypeStruct((NC, 2, NPAD), jnp.float32),
      mesh=mesh,
      compiler_params=cparams,
      scratch_types=[
          pltpu.VMEM_SHARED((NPAD,), jnp.float32),
          pltpu.VMEM_SHARED((NPAD,), jnp.float32),
          pltpu.VMEM_SHARED((NPAD,), jnp.float32),
          pltpu.VMEM((NPC * C,), jnp.float32),
          pltpu.VMEM((NPC,), jnp.float32),
          pltpu.VMEM((112,), jnp.float32),
          pltpu.VMEM((16,), jnp.float32),
          pltpu.VMEM((CHR, ROWW), jnp.int32),
          pltpu.VMEM((CHR, ROWW), jnp.int32),
          pltpu.VMEM((CHR * ROWW * ED,), jnp.float32),
          pltpu.VMEM((CHR * ROWW,), jnp.float32),
          pltpu.VMEM((CHR * ROWW,), jnp.float32),
          pltpu.VMEM((CHR * ROWW,), jnp.float32),
          pltpu.SemaphoreType.DMA,
          pltpu.SemaphoreType.DMA,
          pltpu.SemaphoreType.DMA,
      ],
  )
  k2 = pl.kernel(
      _k2_body,
      out_type=jax.ShapeDtypeStruct((N * C,), jnp.float32),
      mesh=mesh,
      compiler_params=cparams,
      scratch_types=[
          pltpu.VMEM((NPT * C,), jnp.float32),
          pltpu.VMEM((NPT,), jnp.float32),
          pltpu.VMEM((NPT,), jnp.float32),
          pltpu.VMEM((NPT,), jnp.float32),
          pltpu.VMEM((NPT,), jnp.float32),
      ],
  )
  return k1, k2


_KERNS = None


def kernel(x, edge_index, edge_attr, W, b):
  global _KERNS
  if _KERNS is None:
    _KERNS = _make_kernels()
  k1, k2 = _KERNS
  ei3 = edge_index.reshape(2, ROWS, ROWW)
  ea_flat = edge_attr.reshape(-1)
  x_flat = x.reshape(-1)
  hpart = k1(x_flat, ei3, ea_flat, W.reshape(-1), b)
  out_flat = k2(x_flat, hpart)
  return out_flat.reshape(N, C)
